# baseline (speedup 1.0000x reference)
import jax
import jax.numpy as jnp
from jax import lax
from jax.experimental import pallas as pl
from jax.experimental.pallas import tpu as pltpu
from jax.experimental.pallas import tpu_sc as plsc

N_ = 10000
E_ = 320000
D_ = 128
R_ = 400

NC = 2
NS = 16
NW = NC * NS
EPW = E_ // NW
B = 80
NBLK = EPW // B
NP = 10240
CH = NP // NS
SHIFT_MARGIN = 44.0
_QPAIR = (NBLK - 1) // 2

_mesh = plsc.VectorSubcoreMesh(core_axis_name="c", subcore_axis_name="s")
_params = pltpu.CompilerParams(needs_layout_passes=False)


def _lane():
    return lax.broadcasted_iota(jnp.int32, (16,), 0)


def _worker_id():
    return lax.axis_index("c") * NS + lax.axis_index("s")


def _norm_body(ent, rel8, idx3, dst2, norm_out, tmax_out, comp_out,
               denom_out,
               ci0, ci1, srow0, rrow0, trow0, srow1, rrow1, trow1,
               cp0, cp1, normv, tmv, rb0, rb1, didx, mxv, evb, zrow,
               dsp, maxsp,
               gsem0, gsem1, csem0, csem1, wsem0, wsem1):
    c = lax.axis_index("c")
    s_ = lax.axis_index("s")
    wid = c * NS + s_
    lane = _lane()
    for i in range(CH // 16):
        zrow[pl.ds(i * 16, 16)] = jnp.zeros((16,), jnp.float32)
    pltpu.sync_copy(zrow, dsp.at[pl.ds(s_ * CH, CH)])
    pltpu.sync_copy(dst2.at[wid], didx)
    roff = (lane % 8) * R_

    def issue(ci, rb, srow, rrow, trow, sem):
        for i in range(B // 16):
            sl = pl.ds(i * 16, 16)
            rb[0, sl] = ci[2, sl] + roff
        pltpu.async_copy(ent.at[ci.at[0]], srow, sem)
        pltpu.async_copy(rel8.at[rb.at[0]], rrow, sem)
        pltpu.async_copy(ent.at[ci.at[1]], trow, sem)

    def wait3(srow, rrow, trow, sem):
        pltpu.make_async_copy(ent.at[pl.ds(0, B)], srow, sem).wait()
        pltpu.make_async_copy(ent.at[pl.ds(0, B)], rrow, sem).wait()
        pltpu.make_async_copy(ent.at[pl.ds(0, B)], trow, sem).wait()

    def wait_cw(cp, sem):
        pltpu.make_async_copy(cp, comp_out.at[pl.ds(0, B)], sem).wait()

    def block(b, srow, rrow, trow, cp, wsem, m16):
        def grp(g, m16):
            norm16 = jnp.zeros((16,), jnp.float32)
            for j in range(16):
                e = g * 16 + j
                acc = jnp.zeros((16,), jnp.float32)
                for k in range(D_ // 32):
                    sa = pl.ds(k * 32, 16)
                    sb = pl.ds(k * 32 + 16, 16)
                    ua = srow[e, sa] * rrow[e, sa]
                    ub = srow[e, sb] * rrow[e, sb]
                    acc = acc + ua * trow[e, sa] + ub * trow[e, sb]
                    pk = plsc.pack(ua, ub,
                                   format=plsc.PackFormat.INTERLEAVED)
                    cp[e, pl.ds(k * 16, 16)] = plsc.bitcast(pk, jnp.int32)
                rs = jnp.sum(acc)
                norm16 = jnp.where(lane == j, rs, norm16)
            normv[pl.ds(b * B + g * 16, 16)] = norm16
            return jnp.maximum(m16, norm16)

        m16 = lax.fori_loop(0, B // 16, grp, m16)
        pltpu.async_copy(cp, comp_out.at[pl.ds(wid * EPW + b * B, B)], wsem)
        return m16

    pltpu.sync_copy(idx3.at[wid, 0], ci0)
    pltpu.sync_copy(idx3.at[wid, 1], ci1)
    issue(ci0, rb0, srow0, rrow0, trow0, gsem0)
    issue(ci1, rb1, srow1, rrow1, trow1, gsem1)

    def wait_ci(ci, sem):
        pltpu.make_async_copy(idx3.at[wid, 0], ci, sem).wait()

    def pair(q, m16):
        wait3(srow0, rrow0, trow0, gsem0)
        pltpu.async_copy(idx3.at[wid, 2 * q + 2], ci0, csem0)

        @pl.when(q >= 1)
        def _():
            wait_cw(cp0, wsem0)

        m16 = block(2 * q, srow0, rrow0, trow0, cp0, wsem0, m16)
        wait_ci(ci0, csem0)
        issue(ci0, rb0, srow0, rrow0, trow0, gsem0)
        wait3(srow1, rrow1, trow1, gsem1)

        @pl.when(q <= _QPAIR - 2)
        def _():
            pltpu.async_copy(idx3.at[wid, 2 * q + 3], ci1, csem1)

        @pl.when(q >= 1)
        def _():
            wait_cw(cp1, wsem1)

        m16 = block(2 * q + 1, srow1, rrow1, trow1, cp1, wsem1, m16)

        @pl.when(q <= _QPAIR - 2)
        def _():
            wait_ci(ci1, csem1)
            issue(ci1, rb1, srow1, rrow1, trow1, gsem1)

        return m16

    m16 = lax.fori_loop(0, _QPAIR, pair,
                        jnp.full((16,), -jnp.inf, jnp.float32))
    wait3(srow0, rrow0, trow0, gsem0)
    wait_cw(cp0, wsem0)
    m16 = block(NBLK - 1, srow0, rrow0, trow0, cp0, wsem0, m16)
    wait_cw(cp1, wsem1)
    wait_cw(cp0, wsem0)
    tmv[...] = m16
    pltpu.sync_copy(normv, norm_out.at[pl.ds(wid * EPW, EPW)])
    pltpu.sync_copy(tmv, tmax_out.at[wid])
    pltpu.sync_copy(tmv, maxsp.at[s_])
    plsc.subcore_barrier()
    pltpu.sync_copy(maxsp, mxv)
    mx = mxv[0]
    for i in range(1, NS):
        mx = jnp.maximum(mx, mxv[i])
    shift = jnp.max(mx) - SHIFT_MARGIN

    def dblk(b, carry):
        def dgrp(g, carry2):
            sl16 = pl.ds(g * 16, 16)
            evb[sl16] = jnp.exp(normv[pl.ds(b * B + g * 16, 16)] - shift)
            return carry2

        lax.fori_loop(0, B // 16, dgrp, 0)
        pltpu.sync_copy(evb, dsp.at[didx.at[b]], add=True)
        return carry

    lax.fori_loop(0, NBLK, dblk, 0)
    plsc.subcore_barrier()
    pltpu.sync_copy(dsp.at[pl.ds(s_ * CH, CH)],
                    denom_out.at[c, pl.ds(s_ * CH, CH)])


_k1 = pl.kernel(
    _norm_body,
    out_type=(jax.ShapeDtypeStruct((E_,), jnp.float32),
              jax.ShapeDtypeStruct((NW, 16), jnp.float32),
              jax.ShapeDtypeStruct((E_, D_ // 2), jnp.int32),
              jax.ShapeDtypeStruct((NC, NP), jnp.float32)),
    mesh=_mesh,
    compiler_params=_params,
    scratch_types=[
        pltpu.VMEM((3, B), jnp.int32),
        pltpu.VMEM((3, B), jnp.int32),
        pltpu.VMEM((B, D_), jnp.float32),
        pltpu.VMEM((B, D_), jnp.float32),
        pltpu.VMEM((B, D_), jnp.float32),
        pltpu.VMEM((B, D_), jnp.float32),
        pltpu.VMEM((B, D_), jnp.float32),
        pltpu.VMEM((B, D_), jnp.float32),
        pltpu.VMEM((B, D_ // 2), jnp.int32),
        pltpu.VMEM((B, D_ // 2), jnp.int32),
        pltpu.VMEM((EPW,), jnp.float32),
        pltpu.VMEM((16,), jnp.float32),
        pltpu.VMEM((1, B), jnp.int32),
        pltpu.VMEM((1, B), jnp.int32),
        pltpu.VMEM((NBLK, B), jnp.int32),
        pltpu.VMEM((NS, 16), jnp.float32),
        pltpu.VMEM((B,), jnp.float32),
        pltpu.VMEM((CH,), jnp.float32),
        pltpu.VMEM_SHARED((NP,), jnp.float32),
        pltpu.VMEM_SHARED((NS, 16), jnp.float32),
        pltpu.SemaphoreType.DMA,
        pltpu.SemaphoreType.DMA,
        pltpu.SemaphoreType.DMA,
        pltpu.SemaphoreType.DMA,
        pltpu.SemaphoreType.DMA,
        pltpu.SemaphoreType.DMA,
    ],
)


def _agg_body(compin, tmax_in, d0_in, d1_in, combo, neigh_out,
              cb0, cb1, sx0, sx1, ci0, ci1,
              d0v0, d1v0, d0v1, d1v1, co0, co1, tmv, zb,
              nsp, rsem0, rsem1, ssem0, ssem1):
    c = lax.axis_index("c")
    s = lax.axis_index("s")
    wid = c * NS + s
    lane = _lane()

    for j in range(16):
        for k in range(D_ // 16):
            zb[j, pl.ds(k * 16, 16)] = jnp.zeros((16,), jnp.float32)
    for i in range(CH // 16):
        pltpu.sync_copy(zb, nsp.at[pl.ds(s * CH + i * 16, 16)])

    pltpu.sync_copy(tmax_in, tmv)
    m0 = tmv[0]
    m1 = tmv[NS]
    for i in range(1, NS):
        m0 = jnp.maximum(m0, tmv[i])
        m1 = jnp.maximum(m1, tmv[NS + i])
    s0 = jnp.max(m0) - SHIFT_MARGIN
    s1 = jnp.max(m1) - SHIFT_MARGIN
    shift = jnp.maximum(s0, s1)
    f0 = jnp.exp(jnp.full((16,), s0 - shift, jnp.float32))
    f1 = jnp.exp(jnp.full((16,), s1 - shift, jnp.float32))
    plsc.subcore_barrier()

    def issue(b, cb, ci, d0v, d1v, sem):
        pltpu.async_copy(compin.at[pl.ds(wid * EPW + b * B, B)], ci, sem)
        pltpu.async_copy(d0_in.at[cb.at[0]], d0v, sem)
        pltpu.async_copy(d1_in.at[cb.at[0]], d1v, sem)

    def wait3(ci, d0v, d1v, sem):
        pltpu.make_async_copy(compin.at[pl.ds(0, B)], ci, sem).wait()
        pltpu.make_async_copy(d0_in.at[pl.ds(0, B)], d0v, sem).wait()
        pltpu.make_async_copy(d0_in.at[pl.ds(0, B)], d1v, sem).wait()

    def wait_scat(co, sem):
        pltpu.make_async_copy(co, nsp.at[pl.ds(0, B)], sem).wait()

    def block(cb, sx, ci, d0v, d1v, co, sem):
        def grp(g, carry):
            sl16 = pl.ds(g * 16, 16)
            n16 = plsc.bitcast(cb[1, sl16], jnp.float32)
            a16 = jnp.exp(n16 - shift) / (f0 * d0v[sl16] + f1 * d1v[sl16])
            for j in range(16):
                e = g * 16 + j
                aj = jnp.sum(jnp.where(lane == j, a16, 0.0))
                for k in range(D_ // 32):
                    pk = plsc.bitcast(ci[e, pl.ds(k * 16, 16)],
                                      jnp.bfloat16)
                    ua, ub = plsc.unpack(
                        pk, format=plsc.PackFormat.INTERLEAVED)
                    co[e, pl.ds(k * 32, 16)] = ua * aj
                    co[e, pl.ds(k * 32 + 16, 16)] = ub * aj
            return carry

        lax.fori_loop(0, B // 16, grp, 0)
        for i in range(B // 16):
            sx[0, pl.ds(i * 16, 16)] = cb[0, pl.ds(i * 16, 16)]
        pltpu.async_copy(co, nsp.at[sx.at[0]], sem, add=True)

    pltpu.sync_copy(combo.at[wid, 0], cb0)
    issue(0, cb0, ci0, d0v0, d1v0, rsem0)

    def pair(q, carry):
        pltpu.sync_copy(combo.at[wid, 2 * q + 1], cb1)
        issue(2 * q + 1, cb1, ci1, d0v1, d1v1, rsem1)

        @pl.when(q >= 1)
        def _():
            wait_scat(co0, ssem0)

        wait3(ci0, d0v0, d1v0, rsem0)
        block(cb0, sx0, ci0, d0v0, d1v0, co0, ssem0)
        pltpu.sync_copy(combo.at[wid, 2 * q + 2], cb0)
        issue(2 * q + 2, cb0, ci0, d0v0, d1v0, rsem0)

        @pl.when(q >= 1)
        def _():
            wait_scat(co1, ssem1)

        wait3(ci1, d0v1, d1v1, rsem1)
        block(cb1, sx1, ci1, d0v1, d1v1, co1, ssem1)
        return carry

    lax.fori_loop(0, _QPAIR, pair, 0)
    wait_scat(co0, ssem0)
    wait3(ci0, d0v0, d1v0, rsem0)
    block(cb0, sx0, ci0, d0v0, d1v0, co0, ssem0)
    wait_scat(co1, ssem1)
    wait_scat(co0, ssem0)
    plsc.subcore_barrier()
    for i in range(CH // 16):
        pltpu.sync_copy(nsp.at[pl.ds(s * CH + i * 16, 16)],
                        neigh_out.at[c, pl.ds(s * CH + i * 16, 16)])


_k3 = pl.kernel(
    _agg_body,
    out_type=jax.ShapeDtypeStruct((NC, NP, D_), jnp.float32),
    mesh=_mesh,
    compiler_params=_params,
    scratch_types=[
        pltpu.VMEM((2, B), jnp.int32),
        pltpu.VMEM((2, B), jnp.int32),
        pltpu.VMEM((1, B), jnp.int32),
        pltpu.VMEM((1, B), jnp.int32),
        pltpu.VMEM((B, D_ // 2), jnp.int32),
        pltpu.VMEM((B, D_ // 2), jnp.int32),
        pltpu.VMEM((B,), jnp.float32),
        pltpu.VMEM((B,), jnp.float32),
        pltpu.VMEM((B,), jnp.float32),
        pltpu.VMEM((B,), jnp.float32),
        pltpu.VMEM((B, D_), jnp.float32),
        pltpu.VMEM((B, D_), jnp.float32),
        pltpu.VMEM((NW, 16), jnp.float32),
        pltpu.VMEM((16, D_), jnp.float32),
        pltpu.VMEM_SHARED((NP, D_), jnp.float32),
        pltpu.SemaphoreType.DMA,
        pltpu.SemaphoreType.DMA,
        pltpu.SemaphoreType.DMA,
        pltpu.SemaphoreType.DMA,
    ],
)


def _fin_body(p0, p1, w, o):
    o[...] = jnp.tanh(jnp.dot(p0[...] + p1[...], w[...],
                              preferred_element_type=jnp.float32))


def _finish(neigh, w):
    return pl.pallas_call(
        _fin_body,
        grid=(NP // 256,),
        in_specs=[
            pl.BlockSpec((256, D_), lambda i: (i, 0)),
            pl.BlockSpec((256, D_), lambda i: (i, 0)),
            pl.BlockSpec((D_, D_), lambda i: (0, 0)),
        ],
        out_specs=pl.BlockSpec((256, D_), lambda i: (i, 0)),
        out_shape=jax.ShapeDtypeStruct((NP, D_), jnp.float32),
    )(neigh[0], neigh[1], w)


def kernel(ent_emb, rel_emb, neigh_w, src, dst, node_id, rel_id):
    del node_id
    src2 = src.reshape(NW, NBLK, B)
    dst2 = dst.reshape(NW, NBLK, B)
    rel2 = rel_id.reshape(NW, NBLK, B)
    idx3 = jnp.concatenate(
        [src2[:, :, None, :], dst2[:, :, None, :], rel2[:, :, None, :]],
        axis=2)
    rel8 = jnp.tile(rel_emb, (8, 1))
    norm, tmax, comp, denom = _k1(ent_emb, rel8, idx3, dst2)
    combo = jnp.concatenate([
        dst2[:, :, None, :],
        lax.bitcast_convert_type(norm, jnp.int32).reshape(NW, NBLK, 1, B),
    ], axis=2)
    neigh = _k3(comp, tmax, denom[0], denom[1], combo)
    out = _finish(neigh, neigh_w)
    return out[:N_]


# speedup vs baseline: 1.0298x; 1.0298x over previous
import jax
import jax.numpy as jnp
from jax import lax
from jax.experimental import pallas as pl
from jax.experimental.pallas import tpu as pltpu
from jax.experimental.pallas import tpu_sc as plsc

N_ = 10000
E_ = 320000
D_ = 128
R_ = 400

NC = 2
NS = 16
NW = NC * NS
EPW = E_ // NW
B = 80
NBLK = EPW // B
NP = 10240
CH = NP // NS
SHIFT_MARGIN = 44.0
_QPAIR = (NBLK - 1) // 2

_mesh = plsc.VectorSubcoreMesh(core_axis_name="c", subcore_axis_name="s")
_params = pltpu.CompilerParams(needs_layout_passes=False)


def _lane():
    return lax.broadcasted_iota(jnp.int32, (16,), 0)


def _worker_id():
    return lax.axis_index("c") * NS + lax.axis_index("s")


def _norm_body(ent, rel8, idx3, norm_out, tmax_out, comp_out,
               ci0, ci1, srow0, rrow0, trow0, srow1, rrow1, trow1,
               cp0, cp1, normv, tmv, rb0, rb1,
               gsem0, gsem1, csem0, csem1, wsem0, wsem1):
    wid = _worker_id()
    lane = _lane()
    roff = (lane % 8) * R_

    def issue(ci, rb, srow, rrow, trow, sem):
        for i in range(B // 16):
            sl = pl.ds(i * 16, 16)
            rb[0, sl] = ci[2, sl] + roff
        pltpu.async_copy(ent.at[ci.at[0]], srow, sem)
        pltpu.async_copy(rel8.at[rb.at[0]], rrow, sem)
        pltpu.async_copy(ent.at[ci.at[1]], trow, sem)

    def wait3(srow, rrow, trow, sem):
        pltpu.make_async_copy(ent.at[pl.ds(0, B)], srow, sem).wait()
        pltpu.make_async_copy(ent.at[pl.ds(0, B)], rrow, sem).wait()
        pltpu.make_async_copy(ent.at[pl.ds(0, B)], trow, sem).wait()

    def wait_cw(cp, sem):
        pltpu.make_async_copy(cp, comp_out.at[pl.ds(0, B)], sem).wait()

    def block(b, srow, rrow, trow, cp, wsem, m16):
        def grp(g, m16):
            norm16 = jnp.zeros((16,), jnp.float32)
            for j in range(16):
                e = g * 16 + j
                acc = jnp.zeros((16,), jnp.float32)
                for k in range(D_ // 32):
                    sa = pl.ds(k * 32, 16)
                    sb = pl.ds(k * 32 + 16, 16)
                    ua = srow[e, sa] * rrow[e, sa]
                    ub = srow[e, sb] * rrow[e, sb]
                    acc = acc + ua * trow[e, sa] + ub * trow[e, sb]
                    pk = plsc.pack(ua, ub,
                                   format=plsc.PackFormat.INTERLEAVED)
                    cp[e, pl.ds(k * 16, 16)] = plsc.bitcast(pk, jnp.int32)
                rs = jnp.sum(acc)
                norm16 = jnp.where(lane == j, rs, norm16)
            normv[pl.ds(b * B + g * 16, 16)] = norm16
            return jnp.maximum(m16, norm16)

        m16 = lax.fori_loop(0, B // 16, grp, m16)
        pltpu.async_copy(cp, comp_out.at[pl.ds(wid * EPW + b * B, B)], wsem)
        return m16

    pltpu.sync_copy(idx3.at[wid, 0], ci0)
    pltpu.sync_copy(idx3.at[wid, 1], ci1)
    issue(ci0, rb0, srow0, rrow0, trow0, gsem0)
    issue(ci1, rb1, srow1, rrow1, trow1, gsem1)

    def wait_ci(ci, sem):
        pltpu.make_async_copy(idx3.at[wid, 0], ci, sem).wait()

    def pair(q, m16):
        wait3(srow0, rrow0, trow0, gsem0)
        pltpu.async_copy(idx3.at[wid, 2 * q + 2], ci0, csem0)

        @pl.when(q >= 1)
        def _():
            wait_cw(cp0, wsem0)

        m16 = block(2 * q, srow0, rrow0, trow0, cp0, wsem0, m16)
        wait_ci(ci0, csem0)
        issue(ci0, rb0, srow0, rrow0, trow0, gsem0)
        wait3(srow1, rrow1, trow1, gsem1)

        @pl.when(q <= _QPAIR - 2)
        def _():
            pltpu.async_copy(idx3.at[wid, 2 * q + 3], ci1, csem1)

        @pl.when(q >= 1)
        def _():
            wait_cw(cp1, wsem1)

        m16 = block(2 * q + 1, srow1, rrow1, trow1, cp1, wsem1, m16)

        @pl.when(q <= _QPAIR - 2)
        def _():
            wait_ci(ci1, csem1)
            issue(ci1, rb1, srow1, rrow1, trow1, gsem1)

        return m16

    m16 = lax.fori_loop(0, _QPAIR, pair,
                        jnp.full((16,), -jnp.inf, jnp.float32))
    wait3(srow0, rrow0, trow0, gsem0)
    wait_cw(cp0, wsem0)
    m16 = block(NBLK - 1, srow0, rrow0, trow0, cp0, wsem0, m16)
    wait_cw(cp1, wsem1)
    wait_cw(cp0, wsem0)
    tmv[...] = m16
    pltpu.sync_copy(normv, norm_out.at[pl.ds(wid * EPW, EPW)])
    pltpu.sync_copy(tmv, tmax_out.at[wid])


_k1 = pl.kernel(
    _norm_body,
    out_type=(jax.ShapeDtypeStruct((E_,), jnp.float32),
              jax.ShapeDtypeStruct((NW, 16), jnp.float32),
              jax.ShapeDtypeStruct((E_, D_ // 2), jnp.int32)),
    mesh=_mesh,
    compiler_params=_params,
    scratch_types=[
        pltpu.VMEM((3, B), jnp.int32),
        pltpu.VMEM((3, B), jnp.int32),
        pltpu.VMEM((B, D_), jnp.float32),
        pltpu.VMEM((B, D_), jnp.float32),
        pltpu.VMEM((B, D_), jnp.float32),
        pltpu.VMEM((B, D_), jnp.float32),
        pltpu.VMEM((B, D_), jnp.float32),
        pltpu.VMEM((B, D_), jnp.float32),
        pltpu.VMEM((B, D_ // 2), jnp.int32),
        pltpu.VMEM((B, D_ // 2), jnp.int32),
        pltpu.VMEM((EPW,), jnp.float32),
        pltpu.VMEM((16,), jnp.float32),
        pltpu.VMEM((1, B), jnp.int32),
        pltpu.VMEM((1, B), jnp.int32),
        pltpu.SemaphoreType.DMA,
        pltpu.SemaphoreType.DMA,
        pltpu.SemaphoreType.DMA,
        pltpu.SemaphoreType.DMA,
        pltpu.SemaphoreType.DMA,
        pltpu.SemaphoreType.DMA,
    ],
)


def _global_shift(tmv):
    m = tmv[0]
    for i in range(1, NW):
        m = jnp.maximum(m, tmv[i])
    return jnp.max(m) - SHIFT_MARGIN


def _denom_body(norm_in, tmax_in, dst2, denom_out,
                didx, normv, evv, tmv, zrow, dsp):
    c = lax.axis_index("c")
    s = lax.axis_index("s")
    wid = c * NS + s
    for i in range(CH // 16):
        zrow[pl.ds(i * 16, 16)] = jnp.zeros((16,), jnp.float32)
    pltpu.sync_copy(zrow, dsp.at[pl.ds(s * CH, CH)])

    pltpu.sync_copy(tmax_in, tmv)
    shift = _global_shift(tmv)

    pltpu.sync_copy(dst2.at[wid], didx)
    pltpu.sync_copy(norm_in.at[pl.ds(wid * EPW, EPW)], normv)

    def grp(g, carry):
        n16 = normv[pl.ds(g * 16, 16)]
        evv[pl.ds(g * 16, 16)] = jnp.exp(n16 - shift)
        return carry

    lax.fori_loop(0, EPW // 16, grp, 0)
    plsc.subcore_barrier()

    def blk(b, carry):
        pltpu.sync_copy(evv.at[pl.ds(b * B, B)], dsp.at[didx.at[b]], add=True)
        return carry

    lax.fori_loop(0, NBLK, blk, 0)
    plsc.subcore_barrier()
    pltpu.sync_copy(dsp.at[pl.ds(s * CH, CH)],
                    denom_out.at[c, pl.ds(s * CH, CH)])


_k2 = pl.kernel(
    _denom_body,
    out_type=jax.ShapeDtypeStruct((NC, NP), jnp.float32),
    mesh=_mesh,
    compiler_params=_params,
    scratch_types=[
        pltpu.VMEM((NBLK, B), jnp.int32),
        pltpu.VMEM((EPW,), jnp.float32),
        pltpu.VMEM((EPW,), jnp.float32),
        pltpu.VMEM((NW, 16), jnp.float32),
        pltpu.VMEM((CH,), jnp.float32),
        pltpu.VMEM_SHARED((NP,), jnp.float32),
    ],
)


def _agg_body(compin, tmax_in, d0_in, d1_in, combo, neigh_out,
              cb0, cb1, sx0, sx1, ci0, ci1,
              d0v0, d1v0, d0v1, d1v1, co0, co1, tmv, zb,
              nsp, rsem0, rsem1, ssem0, ssem1):
    c = lax.axis_index("c")
    s = lax.axis_index("s")
    wid = c * NS + s
    lane = _lane()

    for j in range(16):
        for k in range(D_ // 16):
            zb[j, pl.ds(k * 16, 16)] = jnp.zeros((16,), jnp.float32)
    for i in range(CH // 16):
        pltpu.sync_copy(zb, nsp.at[pl.ds(s * CH + i * 16, 16)])

    pltpu.sync_copy(tmax_in, tmv)
    shift = _global_shift(tmv)
    plsc.subcore_barrier()

    def issue(b, cb, ci, d0v, d1v, sem):
        pltpu.async_copy(compin.at[pl.ds(wid * EPW + b * B, B)], ci, sem)
        pltpu.async_copy(d0_in.at[cb.at[0]], d0v, sem)
        pltpu.async_copy(d1_in.at[cb.at[0]], d1v, sem)

    def wait3(ci, d0v, d1v, sem):
        pltpu.make_async_copy(compin.at[pl.ds(0, B)], ci, sem).wait()
        pltpu.make_async_copy(d0_in.at[pl.ds(0, B)], d0v, sem).wait()
        pltpu.make_async_copy(d0_in.at[pl.ds(0, B)], d1v, sem).wait()

    def wait_scat(co, sem):
        pltpu.make_async_copy(co, nsp.at[pl.ds(0, B)], sem).wait()

    def block(cb, sx, ci, d0v, d1v, co, sem):
        def grp(g, carry):
            sl16 = pl.ds(g * 16, 16)
            n16 = plsc.bitcast(cb[1, sl16], jnp.float32)
            a16 = jnp.exp(n16 - shift) / (d0v[sl16] + d1v[sl16])
            for j in range(16):
                e = g * 16 + j
                aj = jnp.sum(jnp.where(lane == j, a16, 0.0))
                for k in range(D_ // 32):
                    pk = plsc.bitcast(ci[e, pl.ds(k * 16, 16)],
                                      jnp.bfloat16)
                    ua, ub = plsc.unpack(
                        pk, format=plsc.PackFormat.INTERLEAVED)
                    co[e, pl.ds(k * 32, 16)] = ua * aj
                    co[e, pl.ds(k * 32 + 16, 16)] = ub * aj
            return carry

        lax.fori_loop(0, B // 16, grp, 0)
        for i in range(B // 16):
            sx[0, pl.ds(i * 16, 16)] = cb[0, pl.ds(i * 16, 16)]
        pltpu.async_copy(co, nsp.at[sx.at[0]], sem, add=True)

    pltpu.sync_copy(combo.at[wid, 0], cb0)
    issue(0, cb0, ci0, d0v0, d1v0, rsem0)

    def pair(q, carry):
        pltpu.sync_copy(combo.at[wid, 2 * q + 1], cb1)
        issue(2 * q + 1, cb1, ci1, d0v1, d1v1, rsem1)

        @pl.when(q >= 1)
        def _():
            wait_scat(co0, ssem0)

        wait3(ci0, d0v0, d1v0, rsem0)
        block(cb0, sx0, ci0, d0v0, d1v0, co0, ssem0)
        pltpu.sync_copy(combo.at[wid, 2 * q + 2], cb0)
        issue(2 * q + 2, cb0, ci0, d0v0, d1v0, rsem0)

        @pl.when(q >= 1)
        def _():
            wait_scat(co1, ssem1)

        wait3(ci1, d0v1, d1v1, rsem1)
        block(cb1, sx1, ci1, d0v1, d1v1, co1, ssem1)
        return carry

    lax.fori_loop(0, _QPAIR, pair, 0)
    wait_scat(co0, ssem0)
    wait3(ci0, d0v0, d1v0, rsem0)
    block(cb0, sx0, ci0, d0v0, d1v0, co0, ssem0)
    wait_scat(co1, ssem1)
    wait_scat(co0, ssem0)
    plsc.subcore_barrier()
    for i in range(CH // 16):
        pltpu.sync_copy(nsp.at[pl.ds(s * CH + i * 16, 16)],
                        neigh_out.at[c, pl.ds(s * CH + i * 16, 16)])


_k3 = pl.kernel(
    _agg_body,
    out_type=jax.ShapeDtypeStruct((NC, NP, D_), jnp.float32),
    mesh=_mesh,
    compiler_params=_params,
    scratch_types=[
        pltpu.VMEM((2, B), jnp.int32),
        pltpu.VMEM((2, B), jnp.int32),
        pltpu.VMEM((1, B), jnp.int32),
        pltpu.VMEM((1, B), jnp.int32),
        pltpu.VMEM((B, D_ // 2), jnp.int32),
        pltpu.VMEM((B, D_ // 2), jnp.int32),
        pltpu.VMEM((B,), jnp.float32),
        pltpu.VMEM((B,), jnp.float32),
        pltpu.VMEM((B,), jnp.float32),
        pltpu.VMEM((B,), jnp.float32),
        pltpu.VMEM((B, D_), jnp.float32),
        pltpu.VMEM((B, D_), jnp.float32),
        pltpu.VMEM((NW, 16), jnp.float32),
        pltpu.VMEM((16, D_), jnp.float32),
        pltpu.VMEM_SHARED((NP, D_), jnp.float32),
        pltpu.SemaphoreType.DMA,
        pltpu.SemaphoreType.DMA,
        pltpu.SemaphoreType.DMA,
        pltpu.SemaphoreType.DMA,
    ],
)


def _fin_body(p0, p1, w, o):
    o[...] = jnp.tanh(jnp.dot(p0[...] + p1[...], w[...],
                              preferred_element_type=jnp.float32))


def _finish(neigh, w):
    return pl.pallas_call(
        _fin_body,
        grid=(NP // 256,),
        in_specs=[
            pl.BlockSpec((256, D_), lambda i: (i, 0)),
            pl.BlockSpec((256, D_), lambda i: (i, 0)),
            pl.BlockSpec((D_, D_), lambda i: (0, 0)),
        ],
        out_specs=pl.BlockSpec((256, D_), lambda i: (i, 0)),
        out_shape=jax.ShapeDtypeStruct((NP, D_), jnp.float32),
    )(neigh[0], neigh[1], w)


def kernel(ent_emb, rel_emb, neigh_w, src, dst, node_id, rel_id):
    del node_id
    src2 = src.reshape(NW, NBLK, B)
    dst2 = dst.reshape(NW, NBLK, B)
    rel2 = rel_id.reshape(NW, NBLK, B)
    idx3 = jnp.concatenate(
        [src2[:, :, None, :], dst2[:, :, None, :], rel2[:, :, None, :]],
        axis=2)
    rel8 = jnp.tile(rel_emb, (8, 1))
    norm, tmax, comp = _k1(ent_emb, rel8, idx3)
    denom = _k2(norm, tmax, dst2)
    combo = jnp.concatenate([
        dst2[:, :, None, :],
        lax.bitcast_convert_type(norm, jnp.int32).reshape(NW, NBLK, 1, B),
    ], axis=2)
    neigh = _k3(comp, tmax, denom[0], denom[1], combo)
    out = _finish(neigh, neigh_w)
    return out[:N_]
